# Initial kernel scaffold; baseline (speedup 1.0000x reference)
import functools

import jax
import jax.numpy as jnp
from jax import lax
from jax.experimental import pallas as pl
from jax.experimental.pallas import tpu as pltpu
from jax.experimental.pallas import tpu_sc as plsc

NUM_ENTITIES = 50000
EMBED_DIM = 512
HIDDEN_DIM = 1024
BATCH = 16384

_NC = 2
_NS = 16
_NW = _NC * _NS
_BPW = BATCH // _NW
_CH = 128
_NCHUNK = _BPW // _CH


def _sc_gather_body(entity_hbm, rel_hbm, idxs_hbm, idxr_hbm, idxo_hbm,
                    out_s, out_r, out_o,
                    idxs_v, idxr_v, idxo_v, buf0, buf1,
                    gsem, wsem0, wsem1):
    wid = lax.axis_index("s") * _NC + lax.axis_index("c")
    base = wid * _BPW

    pltpu.sync_copy(idxs_hbm.at[pl.ds(base, _BPW)], idxs_v)
    pltpu.sync_copy(idxr_hbm.at[pl.ds(base, _BPW)], idxr_v)
    pltpu.sync_copy(idxo_hbm.at[pl.ds(base, _BPW)], idxo_v)

    jobs = (
        (entity_hbm, idxs_v, out_s),
        (rel_hbm, idxr_v, out_r),
        (entity_hbm, idxo_v, out_o),
    )
    bufs = (buf0, buf1)
    wsems = (wsem0, wsem1)
    pending = [None, None]
    step = 0
    for table, idx_v, out in jobs:
        for k in range(_NCHUNK):
            slot = step % 2
            if pending[slot] is not None:
                pending[slot].wait()
            buf = bufs[slot]
            pltpu.async_copy(
                table.at[idx_v.at[pl.ds(k * _CH, _CH)]], buf, gsem
            ).wait()
            pending[slot] = pltpu.async_copy(
                buf, out.at[pl.ds(base + k * _CH, _CH)], wsems[slot]
            )
            step += 1
    for p in pending:
        if p is not None:
            p.wait()


_sc_gather = functools.partial(
    pl.kernel,
    out_type=[jax.ShapeDtypeStruct((BATCH, EMBED_DIM), jnp.float32)] * 3,
    mesh=plsc.VectorSubcoreMesh(core_axis_name="c", subcore_axis_name="s"),
    scratch_types=[
        pltpu.VMEM((_BPW,), jnp.int32),
        pltpu.VMEM((_BPW,), jnp.int32),
        pltpu.VMEM((_BPW,), jnp.int32),
        pltpu.VMEM((_CH, EMBED_DIM), jnp.float32),
        pltpu.VMEM((_CH, EMBED_DIM), jnp.float32),
        pltpu.SemaphoreType.DMA,
        pltpu.SemaphoreType.DMA,
        pltpu.SemaphoreType.DMA,
    ],
)(_sc_gather_body)


_BM = 512


def _mlp_body(hs_ref, hr_ref, ho_ref, w1s_ref, w1r_ref, w1o_ref,
              b1_ref, w2t_ref, b2_ref, out_ref):
    acc = jnp.dot(hs_ref[...], w1s_ref[...], preferred_element_type=jnp.float32)
    acc += jnp.dot(hr_ref[...], w1r_ref[...], preferred_element_type=jnp.float32)
    acc += jnp.dot(ho_ref[...], w1o_ref[...], preferred_element_type=jnp.float32)
    hidden = jnp.maximum(acc + b1_ref[...], 0.0)
    out_ref[...] = jnp.sum(hidden * w2t_ref[...], axis=1) + b2_ref[0, 0]


def _mlp(hs, hr, ho, w1s, w1r, w1o, b1, w2t, b2):
    grid = (BATCH // _BM,)
    return pl.pallas_call(
        _mlp_body,
        grid=grid,
        in_specs=[
            pl.BlockSpec((_BM, EMBED_DIM), lambda i: (i, 0)),
            pl.BlockSpec((_BM, EMBED_DIM), lambda i: (i, 0)),
            pl.BlockSpec((_BM, EMBED_DIM), lambda i: (i, 0)),
            pl.BlockSpec((EMBED_DIM, HIDDEN_DIM), lambda i: (0, 0)),
            pl.BlockSpec((EMBED_DIM, HIDDEN_DIM), lambda i: (0, 0)),
            pl.BlockSpec((EMBED_DIM, HIDDEN_DIM), lambda i: (0, 0)),
            pl.BlockSpec((1, HIDDEN_DIM), lambda i: (0, 0)),
            pl.BlockSpec((1, HIDDEN_DIM), lambda i: (0, 0)),
            pl.BlockSpec((1, 1), lambda i: (0, 0)),
        ],
        out_specs=pl.BlockSpec((_BM,), lambda i: (i,)),
        out_shape=jax.ShapeDtypeStruct((BATCH,), jnp.float32),
    )(hs, hr, ho, w1s, w1r, w1o, b1, w2t, b2)


def kernel(entity_emb, triples, rel_emb, fc1, fc1_bias, fc2, fc2_bias):
    idx = triples.astype(jnp.int32)
    idx_s = idx[:, 0]
    idx_r = idx[:, 1]
    idx_o = idx[:, 2]
    hs, hr, ho = _sc_gather(entity_emb, rel_emb, idx_s, idx_r, idx_o)
    w1s = fc1[:EMBED_DIM]
    w1r = fc1[EMBED_DIM:2 * EMBED_DIM]
    w1o = fc1[2 * EMBED_DIM:]
    b1 = fc1_bias.reshape(1, HIDDEN_DIM)
    w2t = fc2.reshape(1, HIDDEN_DIM)
    b2 = fc2_bias.reshape(1, 1)
    return _mlp(hs, hr, ho, w1s, w1r, w1o, b1, w2t, b2)


# speedup vs baseline: 5.8804x; 5.8804x over previous
import functools

import jax
import jax.numpy as jnp
from jax import lax
from jax.experimental import pallas as pl
from jax.experimental.pallas import tpu as pltpu
from jax.experimental.pallas import tpu_sc as plsc

NUM_ENTITIES = 50000
EMBED_DIM = 512
HIDDEN_DIM = 1024
BATCH = 16384

_NC = 2
_NS = 16
_NW = _NC * _NS
_BPW = BATCH // _NW
_CH = 64
_NCHUNK = _BPW // _CH


def _sc_gather_body(entity_hbm, rel_hbm, idxs_hbm, idxr_hbm, idxo_hbm,
                    out_s, out_r, out_o,
                    idxs_v, idxr_v, idxo_v, buf0, buf1,
                    gsem, wsem0, wsem1):
    wid = lax.axis_index("s") * _NC + lax.axis_index("c")
    base = wid * _BPW

    pltpu.sync_copy(idxs_hbm.at[pl.ds(base, _BPW)], idxs_v)
    pltpu.sync_copy(idxr_hbm.at[pl.ds(base, _BPW)], idxr_v)
    pltpu.sync_copy(idxo_hbm.at[pl.ds(base, _BPW)], idxo_v)

    jobs = (
        (entity_hbm, idxs_v, out_s),
        (rel_hbm, idxr_v, out_r),
        (entity_hbm, idxo_v, out_o),
    )
    bufs = (buf0, buf1)
    wsems = (wsem0, wsem1)
    pending = [None, None]
    step = 0
    for table, idx_v, out in jobs:
        for k in range(_NCHUNK):
            slot = step % 2
            if pending[slot] is not None:
                pending[slot].wait()
            buf = bufs[slot]
            pltpu.async_copy(
                table.at[idx_v.at[pl.ds(k * _CH, _CH)]], buf, gsem
            ).wait()
            pending[slot] = pltpu.async_copy(
                buf, out.at[pl.ds(base + k * _CH, _CH)], wsems[slot]
            )
            step += 1
    for p in pending:
        if p is not None:
            p.wait()


@functools.cache
def _sc_gather():
    return functools.partial(
        pl.kernel,
        out_type=[jax.ShapeDtypeStruct((BATCH, EMBED_DIM), jnp.float32)] * 3,
        mesh=plsc.VectorSubcoreMesh(core_axis_name="c", subcore_axis_name="s",
                                    num_cores=_NC, num_subcores=_NS),
        scratch_types=[
            pltpu.VMEM((_BPW,), jnp.int32),
            pltpu.VMEM((_BPW,), jnp.int32),
            pltpu.VMEM((_BPW,), jnp.int32),
            pltpu.VMEM((_CH, EMBED_DIM), jnp.float32),
            pltpu.VMEM((_CH, EMBED_DIM), jnp.float32),
            pltpu.SemaphoreType.DMA,
            pltpu.SemaphoreType.DMA,
            pltpu.SemaphoreType.DMA,
        ],
    )(_sc_gather_body)


_BM = 512


def _mlp_body(hs_ref, hr_ref, ho_ref, w1s_ref, w1r_ref, w1o_ref,
              b1_ref, w2t_ref, b2_ref, out_ref):
    acc = jnp.dot(hs_ref[...], w1s_ref[...], preferred_element_type=jnp.float32)
    acc += jnp.dot(hr_ref[...], w1r_ref[...], preferred_element_type=jnp.float32)
    acc += jnp.dot(ho_ref[...], w1o_ref[...], preferred_element_type=jnp.float32)
    hidden = jnp.maximum(acc + b1_ref[...], 0.0)
    out_ref[...] = jnp.sum(hidden * w2t_ref[...], axis=1) + b2_ref[0, 0]


def _mlp(hs, hr, ho, w1s, w1r, w1o, b1, w2t, b2):
    grid = (BATCH // _BM,)
    return pl.pallas_call(
        _mlp_body,
        grid=grid,
        in_specs=[
            pl.BlockSpec((_BM, EMBED_DIM), lambda i: (i, 0)),
            pl.BlockSpec((_BM, EMBED_DIM), lambda i: (i, 0)),
            pl.BlockSpec((_BM, EMBED_DIM), lambda i: (i, 0)),
            pl.BlockSpec((EMBED_DIM, HIDDEN_DIM), lambda i: (0, 0)),
            pl.BlockSpec((EMBED_DIM, HIDDEN_DIM), lambda i: (0, 0)),
            pl.BlockSpec((EMBED_DIM, HIDDEN_DIM), lambda i: (0, 0)),
            pl.BlockSpec((1, HIDDEN_DIM), lambda i: (0, 0)),
            pl.BlockSpec((1, HIDDEN_DIM), lambda i: (0, 0)),
            pl.BlockSpec((1, 1), lambda i: (0, 0)),
        ],
        out_specs=pl.BlockSpec((_BM,), lambda i: (i,)),
        out_shape=jax.ShapeDtypeStruct((BATCH,), jnp.float32),
    )(hs, hr, ho, w1s, w1r, w1o, b1, w2t, b2)


def kernel(entity_emb, triples, rel_emb, fc1, fc1_bias, fc2, fc2_bias):
    idx = triples.astype(jnp.int32)
    idx_s = idx[:, 0]
    idx_r = idx[:, 1]
    idx_o = idx[:, 2]
    hs, hr, ho = _sc_gather()(entity_emb, rel_emb, idx_s, idx_r, idx_o)
    w1s = fc1[:EMBED_DIM]
    w1r = fc1[EMBED_DIM:2 * EMBED_DIM]
    w1o = fc1[2 * EMBED_DIM:]
    b1 = fc1_bias.reshape(1, HIDDEN_DIM)
    w2t = fc2.reshape(1, HIDDEN_DIM)
    b2 = fc2_bias.reshape(1, 1)
    return _mlp(hs, hr, ho, w1s, w1r, w1o, b1, w2t, b2)
